# Initial kernel scaffold; baseline (speedup 1.0000x reference)
import jax
import jax.numpy as jnp
from jax.experimental import pallas as pl


def kernel(inputs, input_types, neigh_list, es_W1, es_b1, es_W2, es_b2, fs_W1, fs_b1, fs_W2, fs_b2, en_W1, en_b1, en_W2, en_b2):
    raise NotImplementedError("write your pallas kernel here")


# speedup vs baseline: 7.5297x; 7.5297x over previous
import functools
import math

import jax
import jax.numpy as jnp
from jax import lax
from jax.experimental import pallas as pl
from jax.experimental.pallas import tpu as pltpu
from jax.experimental.pallas import tpu_sc as plsc

LENGTH = 10.0
R_CS = 2.0
R_C = 3.0
DIM_SUB = 8

_NC = 2
_NS = 16
_NW = _NC * _NS


def _gather_rows8(table8, idx_flat):
    n = idx_flat.shape[0]
    per_w = n // _NW
    assert per_w * _NW == n
    nch = next(k for k in range(1, per_w + 1)
               if per_w % k == 0 and per_w // k <= 6400
               and (per_w // k) % 8 == 0)
    ch = per_w // nch

    mesh = plsc.VectorSubcoreMesh(core_axis_name="c", subcore_axis_name="s")

    @functools.partial(
        pl.kernel,
        mesh=mesh,
        compiler_params=pltpu.CompilerParams(use_tc_tiling_on_sc=False),
        out_type=jax.ShapeDtypeStruct((n, 8), jnp.float32),
        scratch_types=[
            pltpu.VMEM((ch,), jnp.int32),
            pltpu.VMEM((ch, 8), jnp.float32),
            pltpu.SemaphoreType.DMA,
        ],
    )
    def gk(tab_hbm, idx_hbm, out_hbm, idx_v, rows_v, sem_g):
        wid = lax.axis_index("s") * _NC + lax.axis_index("c")
        base = wid * per_w

        def chunk(c, carry):
            off = base + c * ch
            pltpu.sync_copy(idx_hbm.at[pl.ds(off, ch)], idx_v)
            pltpu.async_copy(tab_hbm.at[idx_v], rows_v, sem_g).wait()
            pltpu.sync_copy(rows_v, out_hbm.at[pl.ds(off, ch)])
            return carry

        lax.fori_loop(0, nch, chunk, 0)

    return gk(table8, idx_flat)


def _mm_small(x, w):
    k = w.shape[0]
    acc = x[:, 0:1] * w[0:1, :]
    for j in range(1, k):
        acc = acc + x[:, j : j + 1] * w[j : j + 1, :]
    return acc


def _tc_body(g_ref, self_ref,

             es_W1, es_b1, es_W2, es_b2,
             fs_W1, fs_b1, fs_W2, fs_b2,
             en_W1, en_b1, en_W2, en_b2,
             out_ref):
    blk = g_ref.shape[0]
    m = g_ref.shape[1] // 8

    v = lax.broadcasted_iota(jnp.int32, (4, 1), 0)
    a = (v // 2).astype(jnp.float32)
    b = (v % 2).astype(jnp.float32)
    pab = jnp.concatenate([a, b], axis=1)
    prev = jnp.concatenate([b, a], axis=1)

    def es_chain(p):
        e1 = jnp.maximum(_mm_small(p, es_W1[...]) + es_b1[...], 0.0)
        return _mm_small(e1, es_W2[...]) + es_b2[...]

    td = es_chain(pab) + es_chain(prev)
    f1 = jnp.maximum(_mm_small(td, fs_W1[...]) + fs_b1[...], 0.0)
    sd = _mm_small(f1, fs_W2[...]) + fs_b2[...]
    w1 = en_W1[...]
    c_tab = _mm_small(sd, w1[0:8, :]) + en_b1[...]
    w9 = w1[8:9, :].reshape(1, 1, 32)

    c00 = c_tab[0:1, :].reshape(1, 1, 32)
    c01 = c_tab[1:2, :].reshape(1, 1, 32)
    c10 = c_tab[2:3, :].reshape(1, 1, 32)
    c11 = c_tab[3:4, :].reshape(1, 1, 32)

    gr = g_ref[...].reshape(blk, m, 8)
    xg = gr[:, :, 0]
    yg = gr[:, :, 1]
    zg = gr[:, :, 2]
    tg = gr[:, :, 3]

    srow = self_ref[...]
    xs = srow[:, 0:1]
    ys = srow[:, 1:2]
    zs = srow[:, 2:3]
    ts = srow[:, 3:4]

    dx = xg - xs
    dy = yg - ys
    dz = zg - zs
    dx = dx - LENGTH * jnp.round(dx * (1.0 / LENGTH))
    dy = dy - LENGTH * jnp.round(dy * (1.0 / LENGTH))
    dz = dz - LENGTH * jnp.round(dz * (1.0 / LENGTH))
    r2 = dx * dx + dy * dy + dz * dz
    r = jnp.sqrt(r2)
    safe_r = jnp.where(r > 1e-12, r, 1.0)
    inv = 1.0 / safe_r
    u = (r - R_CS) / (R_C - R_CS)
    sw = inv * (0.5 * jnp.cos(math.pi * u) + 0.5)
    s = jnp.where(r < R_CS, inv, jnp.where(r < R_C, sw, 0.0))

    ts3 = ts.reshape(blk, 1, 1)
    tn3 = tg.reshape(blk, m, 1)
    base = c00 + ts3 * (c10 - c00)
    coef = (c01 - c00) + ts3 * (c11 + c00 - c10 - c01)
    pre = base + tn3 * coef + s.reshape(blk, m, 1) * w9
    h = jnp.maximum(pre, 0.0)

    h2 = h.reshape(blk * m, 32)
    g2 = jnp.dot(h2, en_W2[...], preferred_element_type=jnp.float32,
                 precision=lax.Precision.HIGHEST)
    g3 = (g2 + en_b2[...] + h2).reshape(blk, m, 32)

    sinv = s * inv
    w_0 = s
    w_x = dx * sinv
    w_y = dy * sinv
    w_z = dz * sinv

    a0 = jnp.sum(g3 * w_0.reshape(blk, m, 1), axis=1)
    a1 = jnp.sum(g3 * w_x.reshape(blk, m, 1), axis=1)
    a2 = jnp.sum(g3 * w_y.reshape(blk, m, 1), axis=1)
    a3 = jnp.sum(g3 * w_z.reshape(blk, m, 1), axis=1)

    rr = lax.broadcasted_iota(jnp.int32, (32, 256), 0)
    cc = lax.broadcasted_iota(jnp.int32, (32, 256), 1)
    e_mat = (rr == cc // 8).astype(jnp.float32)
    rr8 = lax.broadcasted_iota(jnp.int32, (8, 256), 0)
    cc8 = lax.broadcasted_iota(jnp.int32, (8, 256), 1)
    f_mat = (rr8 == cc8 % 8).astype(jnp.float32)

    acc = jnp.zeros((blk, 256), jnp.float32)
    for ai in (a0, a1, a2, a3):
        rep = jnp.dot(ai, e_mat, preferred_element_type=jnp.float32,
                      precision=lax.Precision.HIGHEST)
        til = jnp.dot(ai[:, 0:8], f_mat, preferred_element_type=jnp.float32,
                      precision=lax.Precision.HIGHEST)
        acc = acc + rep * til
    out_ref[...] = acc


def _run_tc(gathered2d, table, weights, blk):
    n_atoms, row = gathered2d.shape
    nb = n_atoms // blk
    assert nb * blk == n_atoms
    wspecs = [pl.BlockSpec(w.shape, lambda i, nd=w.ndim: (0,) * nd)
              for w in weights]
    return pl.pallas_call(
        _tc_body,
        grid=(nb,),
        in_specs=[
            pl.BlockSpec((blk, row), lambda i: (i, 0)),
            pl.BlockSpec((blk, 4), lambda i: (i, 0)),
            *wspecs,
        ],
        out_specs=pl.BlockSpec((blk, 256), lambda i: (i, 0)),
        out_shape=jax.ShapeDtypeStruct((n_atoms, 256), jnp.float32),
        compiler_params=pltpu.CompilerParams(
            dimension_semantics=("arbitrary",),
        ),
    )(gathered2d, table, *weights)


def kernel(inputs, input_types, neigh_list,
           es_W1, es_b1, es_W2, es_b2,
           fs_W1, fs_b1, fs_W2, fs_b2,
           en_W1, en_b1, en_W2, en_b2):
    s_n, p_n, _ = inputs.shape
    m = neigh_list.shape[2]
    n_atoms = s_n * p_n

    table = jnp.concatenate(
        [inputs.reshape(n_atoms, 3),
         input_types.reshape(n_atoms, 1).astype(jnp.float32)], axis=1)
    table8 = jnp.concatenate(
        [table, jnp.zeros((n_atoms, 4), jnp.float32)], axis=1)

    offs = (jnp.arange(s_n, dtype=jnp.int32) * p_n).reshape(s_n, 1, 1)
    idx_flat = (jnp.maximum(neigh_list, 0) + offs).reshape(n_atoms * m)

    gathered = _gather_rows8(table8, idx_flat)
    gathered2d = gathered.reshape(n_atoms, m * 8)

    weights = [es_W1, es_b1.reshape(1, -1), es_W2, es_b2.reshape(1, -1),
               fs_W1, fs_b1.reshape(1, -1), fs_W2, fs_b2.reshape(1, -1),
               en_W1, en_b1.reshape(1, -1), en_W2, en_b2.reshape(1, -1)]

    out = _run_tc(gathered2d, table, weights, blk=200)
    return out.reshape(s_n, p_n, 32, DIM_SUB)


# speedup vs baseline: 17.0284x; 2.2615x over previous
import functools
import math

import jax
import jax.numpy as jnp
from jax import lax
from jax.experimental import pallas as pl
from jax.experimental.pallas import tpu as pltpu
from jax.experimental.pallas import tpu_sc as plsc

LENGTH = 10.0
R_CS = 2.0
R_C = 3.0
DIM_SUB = 8

_NC = 2
_NS = 16
_NW = _NC * _NS


def _gather_rows8(table8, idx_flat):
    n = idx_flat.shape[0]
    per_w = n // _NW
    assert per_w * _NW == n
    nch = next(k for k in range(1, per_w + 1)
               if per_w % k == 0 and per_w // k <= 6400
               and (per_w // k) % 8 == 0)
    ch = per_w // nch

    mesh = plsc.VectorSubcoreMesh(core_axis_name="c", subcore_axis_name="s")

    @functools.partial(
        pl.kernel,
        mesh=mesh,
        compiler_params=pltpu.CompilerParams(use_tc_tiling_on_sc=False),
        out_type=jax.ShapeDtypeStruct((n, 8), jnp.float32),
        scratch_types=[
            pltpu.VMEM((ch,), jnp.int32),
            pltpu.VMEM((ch, 8), jnp.float32),
            pltpu.SemaphoreType.DMA,
        ],
    )
    def gk(tab_hbm, idx_hbm, out_hbm, idx_v, rows_v, sem_g):
        wid = lax.axis_index("s") * _NC + lax.axis_index("c")
        base = wid * per_w

        def chunk(c, carry):
            off = base + c * ch
            pltpu.sync_copy(idx_hbm.at[pl.ds(off, ch)], idx_v)
            pltpu.async_copy(tab_hbm.at[idx_v], rows_v, sem_g).wait()
            pltpu.sync_copy(rows_v, out_hbm.at[pl.ds(off, ch)])
            return carry

        lax.fori_loop(0, nch, chunk, 0)

    return gk(table8, idx_flat)


def _mm_small(x, w):
    k = w.shape[0]
    acc = x[:, 0:1] * w[0:1, :]
    for j in range(1, k):
        acc = acc + x[:, j : j + 1] * w[j : j + 1, :]
    return acc


def _tc_body(g_ref, self_ref,

             es_W1, es_b1, es_W2, es_b2,
             fs_W1, fs_b1, fs_W2, fs_b2,
             en_W1, en_b1, en_W2, en_b2,
             out_ref):
    blk = g_ref.shape[0]
    m = g_ref.shape[1] // 8

    v = lax.broadcasted_iota(jnp.int32, (4, 1), 0)
    a = (v // 2).astype(jnp.float32)
    b = (v % 2).astype(jnp.float32)
    pab = jnp.concatenate([a, b], axis=1)
    prev = jnp.concatenate([b, a], axis=1)

    def es_chain(p):
        e1 = jnp.maximum(_mm_small(p, es_W1[...]) + es_b1[...], 0.0)
        return _mm_small(e1, es_W2[...]) + es_b2[...]

    td = es_chain(pab) + es_chain(prev)
    f1 = jnp.maximum(_mm_small(td, fs_W1[...]) + fs_b1[...], 0.0)
    sd = _mm_small(f1, fs_W2[...]) + fs_b2[...]
    w1 = en_W1[...]
    c_tab = _mm_small(sd, w1[0:8, :]) + en_b1[...]
    w9 = w1[8:9, :].reshape(1, 1, 32)

    c00 = c_tab[0:1, :].reshape(1, 1, 32)
    c01 = c_tab[1:2, :].reshape(1, 1, 32)
    c10 = c_tab[2:3, :].reshape(1, 1, 32)
    c11 = c_tab[3:4, :].reshape(1, 1, 32)

    g = g_ref[...]
    r0 = lax.broadcasted_iota(jnp.int32, (8 * m, 4 * m), 0)
    c0 = lax.broadcasted_iota(jnp.int32, (8 * m, 4 * m), 1)
    sel = (r0 == 8 * (c0 % m) + (c0 // m)).astype(jnp.float32)
    g1 = g.astype(jnp.bfloat16).astype(jnp.float32)
    grem = g - g1
    gmid = grem.astype(jnp.bfloat16).astype(jnp.float32)
    glo = grem - gmid
    planes = (jnp.dot(g1, sel, preferred_element_type=jnp.float32)
              + jnp.dot(gmid, sel, preferred_element_type=jnp.float32)
              + jnp.dot(glo, sel, preferred_element_type=jnp.float32))
    xg = planes[:, 0 * m:1 * m]
    yg = planes[:, 1 * m:2 * m]
    zg = planes[:, 2 * m:3 * m]
    tg = planes[:, 3 * m:4 * m]

    srow = self_ref[...]
    xs = srow[:, 0:1]
    ys = srow[:, 1:2]
    zs = srow[:, 2:3]
    ts = srow[:, 3:4]

    dx = xg - xs
    dy = yg - ys
    dz = zg - zs
    dx = dx - LENGTH * jnp.round(dx * (1.0 / LENGTH))
    dy = dy - LENGTH * jnp.round(dy * (1.0 / LENGTH))
    dz = dz - LENGTH * jnp.round(dz * (1.0 / LENGTH))
    r2 = dx * dx + dy * dy + dz * dz
    r = jnp.sqrt(r2)
    safe_r = jnp.where(r > 1e-12, r, 1.0)
    inv = 1.0 / safe_r
    u = (r - R_CS) / (R_C - R_CS)
    sw = inv * (0.5 * jnp.cos(math.pi * u) + 0.5)
    s = jnp.where(r < R_CS, inv, jnp.where(r < R_C, sw, 0.0))

    ts3 = ts.reshape(blk, 1, 1)
    tn3 = tg.reshape(blk, m, 1)
    base = c00 + ts3 * (c10 - c00)
    coef = (c01 - c00) + ts3 * (c11 + c00 - c10 - c01)
    pre = base + tn3 * coef + s.reshape(blk, m, 1) * w9
    h = jnp.maximum(pre, 0.0)

    h2 = h.reshape(blk * m, 32)
    g2 = jnp.dot(h2, en_W2[...], preferred_element_type=jnp.float32,
                 precision=lax.Precision.HIGHEST)
    g3 = (g2 + en_b2[...] + h2).reshape(blk, m, 32)

    sinv = s * inv
    w_0 = s
    w_x = dx * sinv
    w_y = dy * sinv
    w_z = dz * sinv

    a0 = jnp.sum(g3 * w_0.reshape(blk, m, 1), axis=1)
    a1 = jnp.sum(g3 * w_x.reshape(blk, m, 1), axis=1)
    a2 = jnp.sum(g3 * w_y.reshape(blk, m, 1), axis=1)
    a3 = jnp.sum(g3 * w_z.reshape(blk, m, 1), axis=1)

    rr = lax.broadcasted_iota(jnp.int32, (32, 256), 0)
    cc = lax.broadcasted_iota(jnp.int32, (32, 256), 1)
    e_mat = (rr == cc // 8).astype(jnp.float32)
    rr8 = lax.broadcasted_iota(jnp.int32, (8, 256), 0)
    cc8 = lax.broadcasted_iota(jnp.int32, (8, 256), 1)
    f_mat = (rr8 == cc8 % 8).astype(jnp.float32)

    acc = jnp.zeros((blk, 256), jnp.float32)
    for ai in (a0, a1, a2, a3):
        rep = jnp.dot(ai, e_mat, preferred_element_type=jnp.float32,
                      precision=lax.Precision.HIGHEST)
        til = jnp.dot(ai[:, 0:8], f_mat, preferred_element_type=jnp.float32,
                      precision=lax.Precision.HIGHEST)
        acc = acc + rep * til
    out_ref[...] = acc


def _run_tc(gathered2d, table, weights, blk):
    n_atoms, row = gathered2d.shape
    nb = n_atoms // blk
    assert nb * blk == n_atoms
    wspecs = [pl.BlockSpec(w.shape, lambda i, nd=w.ndim: (0,) * nd)
              for w in weights]
    return pl.pallas_call(
        _tc_body,
        grid=(nb,),
        in_specs=[
            pl.BlockSpec((blk, row), lambda i: (i, 0)),
            pl.BlockSpec((blk, 4), lambda i: (i, 0)),
            *wspecs,
        ],
        out_specs=pl.BlockSpec((blk, 256), lambda i: (i, 0)),
        out_shape=jax.ShapeDtypeStruct((n_atoms, 256), jnp.float32),
        compiler_params=pltpu.CompilerParams(
            dimension_semantics=("arbitrary",),
        ),
    )(gathered2d, table, *weights)


def kernel(inputs, input_types, neigh_list,
           es_W1, es_b1, es_W2, es_b2,
           fs_W1, fs_b1, fs_W2, fs_b2,
           en_W1, en_b1, en_W2, en_b2):
    s_n, p_n, _ = inputs.shape
    m = neigh_list.shape[2]
    n_atoms = s_n * p_n

    table = jnp.concatenate(
        [inputs.reshape(n_atoms, 3),
         input_types.reshape(n_atoms, 1).astype(jnp.float32)], axis=1)
    table8 = jnp.concatenate(
        [table, jnp.zeros((n_atoms, 4), jnp.float32)], axis=1)

    offs = (jnp.arange(s_n, dtype=jnp.int32) * p_n).reshape(s_n, 1, 1)
    idx_flat = (jnp.maximum(neigh_list, 0) + offs).reshape(n_atoms * m)

    gathered = _gather_rows8(table8, idx_flat)
    gathered2d = gathered.reshape(n_atoms, m * 8)

    weights = [es_W1, es_b1.reshape(1, -1), es_W2, es_b2.reshape(1, -1),
               fs_W1, fs_b1.reshape(1, -1), fs_W2, fs_b2.reshape(1, -1),
               en_W1, en_b1.reshape(1, -1), en_W2, en_b2.reshape(1, -1)]

    out = _run_tc(gathered2d, table, weights, blk=400)
    return out.reshape(s_n, p_n, 32, DIM_SUB)


# speedup vs baseline: 25.7200x; 1.5104x over previous
import functools
import math

import jax
import jax.numpy as jnp
from jax import lax
from jax.experimental import pallas as pl
from jax.experimental.pallas import tpu as pltpu
from jax.experimental.pallas import tpu_sc as plsc

LENGTH = 10.0
R_CS = 2.0
R_C = 3.0
DIM_SUB = 8

_NC = 2
_NS = 16
_NW = _NC * _NS


def _gather_rows8(table8, idx_flat):
    n = idx_flat.shape[0]
    per_w = n // _NW
    assert per_w * _NW == n
    nch = next(k for k in range(1, per_w + 1)
               if per_w % k == 0 and per_w // k <= 6400
               and (per_w // k) % 8 == 0)
    ch = per_w // nch

    mesh = plsc.VectorSubcoreMesh(core_axis_name="c", subcore_axis_name="s")

    @functools.partial(
        pl.kernel,
        mesh=mesh,
        compiler_params=pltpu.CompilerParams(use_tc_tiling_on_sc=False),
        out_type=jax.ShapeDtypeStruct((n, 8), jnp.float32),
        scratch_types=[
            pltpu.VMEM((ch,), jnp.int32),
            pltpu.VMEM((ch, 8), jnp.float32),
            pltpu.SemaphoreType.DMA,
        ],
    )
    def gk(tab_hbm, idx_hbm, out_hbm, idx_v, rows_v, sem_g):
        wid = lax.axis_index("s") * _NC + lax.axis_index("c")
        base = wid * per_w

        def chunk(c, carry):
            off = base + c * ch
            pltpu.sync_copy(idx_hbm.at[pl.ds(off, ch)], idx_v)
            pltpu.async_copy(tab_hbm.at[idx_v], rows_v, sem_g).wait()
            pltpu.sync_copy(rows_v, out_hbm.at[pl.ds(off, ch)])
            return carry

        lax.fori_loop(0, nch, chunk, 0)

    return gk(table8, idx_flat)


def _mm_small(x, w):
    k = w.shape[0]
    acc = x[:, 0:1] * w[0:1, :]
    for j in range(1, k):
        acc = acc + x[:, j : j + 1] * w[j : j + 1, :]
    return acc


def _tc_body(g_ref, self_ref,

             es_W1, es_b1, es_W2, es_b2,
             fs_W1, fs_b1, fs_W2, fs_b2,
             en_W1, en_b1, en_W2, en_b2,
             out_ref):
    blk = g_ref.shape[0]
    m = g_ref.shape[1] // 8

    v = lax.broadcasted_iota(jnp.int32, (4, 1), 0)
    a = (v // 2).astype(jnp.float32)
    b = (v % 2).astype(jnp.float32)
    pab = jnp.concatenate([a, b], axis=1)
    prev = jnp.concatenate([b, a], axis=1)

    def es_chain(p):
        e1 = jnp.maximum(_mm_small(p, es_W1[...]) + es_b1[...], 0.0)
        return _mm_small(e1, es_W2[...]) + es_b2[...]

    td = es_chain(pab) + es_chain(prev)
    f1 = jnp.maximum(_mm_small(td, fs_W1[...]) + fs_b1[...], 0.0)
    sd = _mm_small(f1, fs_W2[...]) + fs_b2[...]
    w1 = en_W1[...]
    c_tab = _mm_small(sd, w1[0:8, :]) + en_b1[...]
    w9 = w1[8:9, :].reshape(1, 1, 32)

    c00 = c_tab[0:1, :].reshape(1, 1, 32)
    c01 = c_tab[1:2, :].reshape(1, 1, 32)
    c10 = c_tab[2:3, :].reshape(1, 1, 32)
    c11 = c_tab[3:4, :].reshape(1, 1, 32)

    g = g_ref[...]
    r0 = lax.broadcasted_iota(jnp.int32, (8 * m, 4 * m), 0)
    c0 = lax.broadcasted_iota(jnp.int32, (8 * m, 4 * m), 1)
    sel = (r0 == 8 * (c0 % m) + (c0 // m)).astype(jnp.float32)
    g1 = g.astype(jnp.bfloat16).astype(jnp.float32)
    grem = g - g1
    gmid = grem.astype(jnp.bfloat16).astype(jnp.float32)
    glo = grem - gmid
    planes = (jnp.dot(g1, sel, preferred_element_type=jnp.float32)
              + jnp.dot(gmid, sel, preferred_element_type=jnp.float32)
              + jnp.dot(glo, sel, preferred_element_type=jnp.float32))
    xg = planes[:, 0 * m:1 * m]
    yg = planes[:, 1 * m:2 * m]
    zg = planes[:, 2 * m:3 * m]
    tg = planes[:, 3 * m:4 * m]

    srow = self_ref[...]
    xs = srow[:, 0:1]
    ys = srow[:, 1:2]
    zs = srow[:, 2:3]
    ts = srow[:, 3:4]

    dx = xg - xs
    dy = yg - ys
    dz = zg - zs
    dx = dx - LENGTH * jnp.round(dx * (1.0 / LENGTH))
    dy = dy - LENGTH * jnp.round(dy * (1.0 / LENGTH))
    dz = dz - LENGTH * jnp.round(dz * (1.0 / LENGTH))
    r2 = dx * dx + dy * dy + dz * dz
    r = jnp.sqrt(r2)
    safe_r = jnp.where(r > 1e-12, r, 1.0)
    inv = 1.0 / safe_r
    u = (r - R_CS) / (R_C - R_CS)
    sw = inv * (0.5 * jnp.cos(math.pi * u) + 0.5)
    s = jnp.where(r < R_CS, inv, jnp.where(r < R_C, sw, 0.0))

    ts3 = ts.reshape(blk, 1, 1)
    tn3 = tg.reshape(blk, m, 1)
    base = c00 + ts3 * (c10 - c00)
    coef = (c01 - c00) + ts3 * (c11 + c00 - c10 - c01)
    sb = s.astype(jnp.bfloat16).astype(jnp.float32)
    w9b = w9.astype(jnp.bfloat16).astype(jnp.float32)
    pre = base + tn3 * coef + sb.reshape(blk, m, 1) * w9b
    h = jnp.maximum(pre, 0.0)

    h2 = h.reshape(blk * m, 32)
    g2 = jnp.dot(h2, en_W2[...], preferred_element_type=jnp.float32)
    g3 = (g2 + en_b2[...] + h2).reshape(blk, m, 32)

    sinv = s * inv
    w_0 = s
    w_x = dx * sinv
    w_y = dy * sinv
    w_z = dz * sinv

    a0 = jnp.sum(g3 * w_0.reshape(blk, m, 1), axis=1)
    a1 = jnp.sum(g3 * w_x.reshape(blk, m, 1), axis=1)
    a2 = jnp.sum(g3 * w_y.reshape(blk, m, 1), axis=1)
    a3 = jnp.sum(g3 * w_z.reshape(blk, m, 1), axis=1)

    rr = lax.broadcasted_iota(jnp.int32, (32, 256), 0)
    cc = lax.broadcasted_iota(jnp.int32, (32, 256), 1)
    e_mat = (rr == cc // 8).astype(jnp.float32)
    rr8 = lax.broadcasted_iota(jnp.int32, (8, 256), 0)
    cc8 = lax.broadcasted_iota(jnp.int32, (8, 256), 1)
    f_mat = (rr8 == cc8 % 8).astype(jnp.float32)

    acc = jnp.zeros((blk, 256), jnp.float32)
    for ai in (a0, a1, a2, a3):
        rep = jnp.dot(ai, e_mat, preferred_element_type=jnp.float32,
                      precision=lax.Precision.HIGHEST)
        til = jnp.dot(ai[:, 0:8], f_mat, preferred_element_type=jnp.float32,
                      precision=lax.Precision.HIGHEST)
        acc = acc + rep * til
    out_ref[...] = acc


def _run_tc(gathered2d, table, weights, blk):
    n_atoms, row = gathered2d.shape
    nb = n_atoms // blk
    assert nb * blk == n_atoms
    wspecs = [pl.BlockSpec(w.shape, lambda i, nd=w.ndim: (0,) * nd)
              for w in weights]
    return pl.pallas_call(
        _tc_body,
        grid=(nb,),
        in_specs=[
            pl.BlockSpec((blk, row), lambda i: (i, 0)),
            pl.BlockSpec((blk, 4), lambda i: (i, 0)),
            *wspecs,
        ],
        out_specs=pl.BlockSpec((blk, 256), lambda i: (i, 0)),
        out_shape=jax.ShapeDtypeStruct((n_atoms, 256), jnp.float32),
        compiler_params=pltpu.CompilerParams(
            dimension_semantics=("arbitrary",),
        ),
    )(gathered2d, table, *weights)


def kernel(inputs, input_types, neigh_list,
           es_W1, es_b1, es_W2, es_b2,
           fs_W1, fs_b1, fs_W2, fs_b2,
           en_W1, en_b1, en_W2, en_b2):
    s_n, p_n, _ = inputs.shape
    m = neigh_list.shape[2]
    n_atoms = s_n * p_n

    table = jnp.concatenate(
        [inputs.reshape(n_atoms, 3),
         input_types.reshape(n_atoms, 1).astype(jnp.float32)], axis=1)
    table8 = jnp.concatenate(
        [table, jnp.zeros((n_atoms, 4), jnp.float32)], axis=1)

    offs = (jnp.arange(s_n, dtype=jnp.int32) * p_n).reshape(s_n, 1, 1)
    idx_flat = (jnp.maximum(neigh_list, 0) + offs).reshape(n_atoms * m)

    gathered = _gather_rows8(table8, idx_flat)
    gathered2d = gathered.reshape(n_atoms, m * 8)

    weights = [es_W1, es_b1.reshape(1, -1), es_W2, es_b2.reshape(1, -1),
               fs_W1, fs_b1.reshape(1, -1), fs_W2, fs_b2.reshape(1, -1),
               en_W1, en_b1.reshape(1, -1), en_W2, en_b2.reshape(1, -1)]

    out = _run_tc(gathered2d, table, weights, blk=400)
    return out.reshape(s_n, p_n, 32, DIM_SUB)
